# Initial kernel scaffold; baseline (speedup 1.0000x reference)
import jax
import jax.numpy as jnp
from jax.experimental import pallas as pl


def kernel(x, pe):
    raise NotImplementedError("write your pallas kernel here")


# speedup vs baseline: 4.0505x; 4.0505x over previous
import functools

import jax
import jax.numpy as jnp
import numpy as np
from jax import lax
from jax.experimental import pallas as pl
from jax.experimental.pallas import tpu as pltpu
from jax.experimental.pallas import tpu_sc as plsc

RESOLUTION = 1000
D = 64
B = 16384
S = 26
N = B * S

NC = 2
NS = 16
NW = NC * NS
PER_W = N // NW
CHUNK = 1024
NCHUNK = PER_W // CHUNK
SUB = 128
NSUB = CHUNK // SUB
LANES = 16

CLIP_LO = np.float32(1.0 / RESOLUTION)
SCALE = np.float32(RESOLUTION)


def _body(x_hbm, pe_hbm, out_hbm, x_v, idx_v, rows_v, sem):
    wid = lax.axis_index("s") * NC + lax.axis_index("c")
    base = wid * PER_W

    def chunk_body(c, carry):
        row0 = base + c * CHUNK
        pltpu.sync_copy(x_hbm.at[pl.ds(row0, CHUNK)], x_v)

        def idx_body(i, carry2):
            off = i * LANES
            v = x_v[pl.ds(off, LANES)]
            t = jnp.maximum(v, CLIP_LO) * SCALE
            magic = np.float32(8388608.0)
            r = (t + magic) - magic
            idx_v[pl.ds(off, LANES)] = r.astype(jnp.int32) - 1
            return carry2

        lax.fori_loop(0, CHUNK // LANES, idx_body, 0)

        copies = [
            pltpu.async_copy(
                pe_hbm.at[idx_v.at[pl.ds(j * SUB, SUB)]],
                rows_v.at[pl.ds(j * SUB, SUB)],
                sem,
            )
            for j in range(NSUB)
        ]
        for cp in copies:
            cp.wait()

        pltpu.sync_copy(rows_v, out_hbm.at[pl.ds(row0, CHUNK)])
        return carry

    lax.fori_loop(0, NCHUNK, chunk_body, 0)


@jax.jit
def _encode(x_flat, pe):
    mesh = plsc.VectorSubcoreMesh(
        core_axis_name="c", subcore_axis_name="s", num_cores=NC, num_subcores=NS
    )
    return pl.kernel(
        _body,
        out_type=jax.ShapeDtypeStruct((N, D), jnp.float32),
        mesh=mesh,
        scratch_types=[
            pltpu.VMEM((CHUNK,), jnp.float32),
            pltpu.VMEM((CHUNK,), jnp.int32),
            pltpu.VMEM((CHUNK, D), jnp.float32),
            pltpu.SemaphoreType.DMA,
        ],
        compiler_params=pltpu.CompilerParams(use_tc_tiling_on_sc=False),
    )(x_flat, pe)


def kernel(x, pe):
    out = _encode(x.reshape(N), pe)
    return out.reshape(B, S, D)
